# baseline (speedup 1.0000x reference)
import functools

import jax
import jax.numpy as jnp
from jax import lax
from jax.experimental import pallas as pl
from jax.experimental.pallas import tpu as pltpu
from jax.experimental.pallas import tpu_sc as plsc

N = 100000
D = 128
G = 512

NC = 2
NS = 16
HALF = D // NC
R = 512
NFULL = N // R
TAIL = N - NFULL * R
NB = NFULL // NS
EXTRA = NFULL - NB * NS
SEG_PER_TILE = G // NS
CH = R // 128

_mesh = plsc.VectorSubcoreMesh(core_axis_name="c", subcore_axis_name="s")


@functools.partial(
    pl.kernel,
    out_type=jax.ShapeDtypeStruct((G, D), jnp.float32),
    mesh=_mesh,
    compiler_params=pltpu.CompilerParams(use_tc_tiling_on_sc=False),
    scratch_types=[
        pltpu.VMEM((2, R, HALF), jnp.float32),
        pltpu.VMEM((2, CH, 128), jnp.int32),
        pltpu.VMEM((32,), jnp.int32),
        pltpu.VMEM((128, 16), jnp.float32),
        pltpu.VMEM((CH, 128), jnp.int32),
        pltpu.VMEM((G, HALF), jnp.float32),
        pltpu.VMEM((SEG_PER_TILE, HALF), jnp.float32),
        pltpu.VMEM((SEG_PER_TILE, 16), jnp.float32),
        pltpu.VMEM_SHARED((G, HALF), jnp.float32),
        pltpu.VMEM_SHARED((G, 16), jnp.float32),
        pltpu.SemaphoreType.DMA,
        pltpu.SemaphoreType.DMA,
        pltpu.SemaphoreType.DMA,
        pltpu.SemaphoreType.DMA,
        pltpu.SemaphoreType.DMA,
    ],
)
def _pool_kernel(x_hbm, b_hbm, z_hbm, ones_hbm, iota_hbm, out_hbm,
                 xb, idxb, idx_t, onesv, iotav, accl, accv, cntv,
                 acc_sh, cnt_sh, sx0, sx1, si0, si1, ssc):
    c = lax.axis_index("c")
    s = lax.axis_index("s")
    col0 = c * HALF
    seg0 = s * SEG_PER_TILE
    sx = (sx0, sx1)
    si = (si0, si1)
    zvec = jnp.zeros((16,), jnp.float32)

    def start_load(kblk, b):
        r0 = (s + NS * kblk) * R
        pltpu.async_copy(x_hbm.at[pl.ds(r0, R), pl.ds(col0, HALF)],
                         xb.at[b], sx[b])
        for m in range(CH):
            pltpu.async_copy(b_hbm.at[pl.ds(r0 + 128 * m, 128)],
                             idxb.at[b, m], si[b])

    def wait_load(b):
        pltpu.make_async_copy(x_hbm.at[pl.ds(0, R), pl.ds(0, HALF)],
                              xb.at[b], sx[b]).wait()
        for m in range(CH):
            pltpu.make_async_copy(b_hbm.at[pl.ds(0, 128)],
                                  idxb.at[b, m], si[b]).wait()

    def accum_rows(b, m, u):
        idvec = idxb[b, m, pl.ds(16 * u, 16)]
        for r in range(16):
            row = 128 * m + 16 * u + r
            sid = idvec[r]
            for j in range(HALF // 16):
                xv = xb[b, row, pl.ds(16 * j, 16)]
                plsc.addupdate(accl.at[sid, pl.ds(16 * j, 16)], xv)

    def process_buf(b):
        handles = [pltpu.async_copy(onesv, cnt_sh.at[idxb.at[b, m]], ssc,
                                    add=True)
                   for m in range(CH)]
        for m in range(CH):
            lax.fori_loop(0, 8, lambda u, _, b=b, m=m:
                          (accum_rows(b, m, u), _)[1], 0)
        for h in handles:
            h.wait()

    start_load(0, 0)
    start_load(1, 1)
    pltpu.sync_copy(z_hbm, acc_sh.at[pl.ds(seg0, SEG_PER_TILE)])
    pltpu.sync_copy(z_hbm.at[:, pl.ds(0, 16)],
                    cnt_sh.at[pl.ds(seg0, SEG_PER_TILE)])
    pltpu.sync_copy(ones_hbm, onesv)
    pltpu.sync_copy(iota_hbm, iotav)

    def zrow(i, carry):
        for j in range(HALF // 16):
            accl[i, pl.ds(16 * j, 16)] = zvec
        return carry

    lax.fori_loop(0, G, zrow, 0)
    plsc.subcore_barrier()

    def blk_body(g, carry):
        for b in range(2):
            wait_load(b)
            process_buf(b)
            start_load(2 * g + b + 2, b)
        return carry

    lax.fori_loop(0, NB // 2 - 1, blk_body, 0)
    for b in range(2):
        wait_load(b)
        process_buf(b)

    @pl.when(s < EXTRA)
    def _extra():
        r0 = (NB * NS + s) * R
        pltpu.sync_copy(x_hbm.at[pl.ds(r0, R), pl.ds(col0, HALF)], xb.at[0])
        for m in range(CH):
            pltpu.sync_copy(b_hbm.at[pl.ds(r0 + 128 * m, 128)], idxb.at[0, m])
        process_buf(0)

    @pl.when(s == EXTRA)
    def _tail():
        r0 = NFULL * R
        pltpu.sync_copy(x_hbm.at[pl.ds(r0, TAIL), pl.ds(col0, HALF)],
                        xb.at[0, pl.ds(0, TAIL)])
        pltpu.sync_copy(b_hbm.at[pl.ds(r0, 128)], idxb.at[0, 0])
        pltpu.sync_copy(onesv, cnt_sh.at[idxb.at[0, 0]], add=True)
        lax.fori_loop(0, 8, lambda u, carry: (accum_rows(0, 0, u), carry)[1],
                      0)
        pltpu.sync_copy(b_hbm.at[pl.ds(r0 + 128, 32)], idx_t)
        pltpu.sync_copy(onesv.at[pl.ds(0, 32)], cnt_sh.at[idx_t], add=True)

        def tail_grp(u, carry):
            idvec = idx_t[pl.ds(16 * u, 16)]
            for r in range(16):
                row = 128 + 16 * u + r
                sid = idvec[r]
                for j in range(HALF // 16):
                    xv = xb[0, row, pl.ds(16 * j, 16)]
                    plsc.addupdate(accl.at[sid, pl.ds(16 * j, 16)], xv)
            return carry

        lax.fori_loop(0, 2, tail_grp, 0)

    handles = [pltpu.async_copy(accl.at[pl.ds(128 * q, 128)],
                                acc_sh.at[iotav.at[q]], ssc, add=True)
               for q in range(CH)]
    for h in handles:
        h.wait()
    plsc.subcore_barrier()

    pltpu.sync_copy(acc_sh.at[pl.ds(seg0, SEG_PER_TILE)], accv)
    pltpu.sync_copy(cnt_sh.at[pl.ds(seg0, SEG_PER_TILE)], cntv)
    for i in range(SEG_PER_TILE):
        inv = 1.0 / jnp.maximum(cntv[i, :], 1.0)
        for j in range(HALF // 16):
            accv[i, pl.ds(16 * j, 16)] = accv[i, pl.ds(16 * j, 16)] * inv
    pltpu.sync_copy(accv,
                    out_hbm.at[pl.ds(seg0, SEG_PER_TILE), pl.ds(col0, HALF)])


def kernel(x, batch):
    zeros = jnp.zeros((SEG_PER_TILE, HALF), jnp.float32)
    ones = jnp.ones((128, 16), jnp.float32)
    iota = jnp.arange(G, dtype=jnp.int32).reshape(CH, 128)
    return _pool_kernel(x, batch, zeros, ones, iota)


# speedup vs baseline: 1.7277x; 1.7277x over previous
import functools

import jax
import jax.numpy as jnp
from jax import lax
from jax.experimental import pallas as pl
from jax.experimental.pallas import tpu as pltpu
from jax.experimental.pallas import tpu_sc as plsc

N = 100000
D = 128
G = 512

NC = 2
NS = 16
HALF = D // NC
R = 512
NFULL = N // R
TAIL = N - NFULL * R
NB = NFULL // NS
EXTRA = NFULL - NB * NS
SEG_PER_TILE = G // NS
CH = R // 128
NSTREAM = 2

_mesh = plsc.VectorSubcoreMesh(core_axis_name="c", subcore_axis_name="s")


@functools.partial(
    pl.kernel,
    out_type=jax.ShapeDtypeStruct((G, D), jnp.float32),
    mesh=_mesh,
    compiler_params=pltpu.CompilerParams(use_tc_tiling_on_sc=False,
                                         needs_layout_passes=False),
    scratch_types=[
        pltpu.VMEM((2, R, HALF), jnp.float32),
        pltpu.VMEM((2, CH, 128), jnp.int32),
        pltpu.VMEM((32,), jnp.int32),
        pltpu.VMEM((CH, 128), jnp.int32),
        pltpu.VMEM((G, HALF), jnp.float32),
        pltpu.VMEM((G, 16), jnp.float32),
        pltpu.VMEM((SEG_PER_TILE, HALF), jnp.float32),
        pltpu.VMEM((SEG_PER_TILE, 16), jnp.float32),
        pltpu.VMEM_SHARED((G, HALF), jnp.float32),
        pltpu.VMEM_SHARED((G, 16), jnp.float32),
        pltpu.SemaphoreType.DMA,
        pltpu.SemaphoreType.DMA,
        pltpu.SemaphoreType.DMA,
        pltpu.SemaphoreType.DMA,
        pltpu.SemaphoreType.DMA,
    ],
)
def _pool_kernel(x_hbm, b_hbm, z_hbm, iota_hbm, out_hbm,
                 xb, idxb, idx_t, iotav, accl, cntl, accv, cntv,
                 acc_sh, cnt_sh, sx0, sx1, si0, si1, ssc):
    c = lax.axis_index("c")
    s = lax.axis_index("s")
    col0 = c * HALF
    seg0 = s * SEG_PER_TILE
    sx = (sx0, sx1)
    si = (si0, si1)
    zvec = jnp.zeros((16,), jnp.float32)
    full16 = jnp.full((16,), 16.0, jnp.float32)
    ones16 = jnp.ones((16,), jnp.float32)

    def start_load(kblk, b):
        r0 = (s + NS * kblk) * R
        pltpu.async_copy(x_hbm.at[pl.ds(r0, R), pl.ds(col0, HALF)],
                         xb.at[b], sx[b])
        for m in range(CH):
            pltpu.async_copy(b_hbm.at[pl.ds(r0 + 128 * m, 128)],
                             idxb.at[b, m], si[b])

    def wait_load(b):
        pltpu.make_async_copy(x_hbm.at[pl.ds(0, R), pl.ds(0, HALF)],
                              xb.at[b], sx[b]).wait()
        for m in range(CH):
            pltpu.make_async_copy(b_hbm.at[pl.ds(0, 128)],
                                  idxb.at[b, m], si[b]).wait()

    def group_body(idvec, row0, b, xmode):
        sid0 = idvec[0]
        uniform = jnp.all(idvec == sid0)

        @pl.when(uniform)
        def _u():
            plsc.addupdate(cntl.at[sid0], full16)
            if xmode:
                for j in range(HALF // 16):
                    acc = xb[b, row0, pl.ds(16 * j, 16)]
                    for r in range(1, 16):
                        acc = acc + xb[b, row0 + r, pl.ds(16 * j, 16)]
                    plsc.addupdate(accl.at[sid0, pl.ds(16 * j, 16)], acc)

        @pl.when(jnp.logical_not(uniform))
        def _n():
            for r in range(16):
                sid = idvec[r]
                plsc.addupdate(cntl.at[sid], ones16)
                if xmode:
                    for j in range(HALF // 16):
                        xv = xb[b, row0 + r, pl.ds(16 * j, 16)]
                        plsc.addupdate(accl.at[sid, pl.ds(16 * j, 16)], xv)

    def process_buf(b):
        handles = [pltpu.async_copy(xb.at[b, pl.ds(128 * m, 128)],
                                    acc_sh.at[idxb.at[b, m]], ssc, add=True)
                   for m in range(NSTREAM)]
        for m in range(CH):
            xmode = m >= NSTREAM

            def grp(u, carry, b=b, m=m, xmode=xmode):
                idvec = idxb[b, m, pl.ds(16 * u, 16)]
                group_body(idvec, 128 * m + 16 * u, b, xmode)
                return carry

            lax.fori_loop(0, 8, grp, 0)
        for h in handles:
            h.wait()

    start_load(0, 0)
    start_load(1, 1)
    pltpu.sync_copy(z_hbm, acc_sh.at[pl.ds(seg0, SEG_PER_TILE)])
    pltpu.sync_copy(z_hbm.at[:, pl.ds(0, 16)],
                    cnt_sh.at[pl.ds(seg0, SEG_PER_TILE)])
    pltpu.sync_copy(iota_hbm, iotav)

    def zrow(i, carry):
        for j in range(HALF // 16):
            accl[i, pl.ds(16 * j, 16)] = zvec
        cntl[i, pl.ds(0, 16)] = zvec
        return carry

    lax.fori_loop(0, G, zrow, 0)
    plsc.subcore_barrier()

    def blk_body(g, carry):
        for b in range(2):
            wait_load(b)
            process_buf(b)
            start_load(2 * g + b + 2, b)
        return carry

    lax.fori_loop(0, NB // 2 - 1, blk_body, 0)
    for b in range(2):
        wait_load(b)
        process_buf(b)

    @pl.when(s < EXTRA)
    def _extra():
        r0 = (NB * NS + s) * R
        pltpu.sync_copy(x_hbm.at[pl.ds(r0, R), pl.ds(col0, HALF)], xb.at[0])
        for m in range(CH):
            pltpu.sync_copy(b_hbm.at[pl.ds(r0 + 128 * m, 128)], idxb.at[0, m])
        process_buf(0)

    @pl.when(s == EXTRA)
    def _tail():
        r0 = NFULL * R
        pltpu.sync_copy(x_hbm.at[pl.ds(r0, TAIL), pl.ds(col0, HALF)],
                        xb.at[0, pl.ds(0, TAIL)])
        pltpu.sync_copy(b_hbm.at[pl.ds(r0, 128)], idxb.at[0, 0])
        pltpu.sync_copy(b_hbm.at[pl.ds(r0 + 128, 32)], idx_t)

        def tgrp(u, carry):
            idvec = idxb[0, 0, pl.ds(16 * u, 16)]
            group_body(idvec, 16 * u, 0, True)
            return carry

        lax.fori_loop(0, 8, tgrp, 0)

        def tgrp2(u, carry):
            idvec = idx_t[pl.ds(16 * u, 16)]
            group_body(idvec, 128 + 16 * u, 0, True)
            return carry

        lax.fori_loop(0, 2, tgrp2, 0)

    handles = [pltpu.async_copy(accl.at[pl.ds(128 * q, 128)],
                                acc_sh.at[iotav.at[q]], ssc, add=True)
               for q in range(CH)]
    handles += [pltpu.async_copy(cntl.at[pl.ds(128 * q, 128)],
                                 cnt_sh.at[iotav.at[q]], ssc, add=True)
                for q in range(CH)]
    for h in handles:
        h.wait()
    plsc.subcore_barrier()

    pltpu.sync_copy(acc_sh.at[pl.ds(seg0, SEG_PER_TILE)], accv)
    pltpu.sync_copy(cnt_sh.at[pl.ds(seg0, SEG_PER_TILE)], cntv)
    for i in range(SEG_PER_TILE):
        inv = 1.0 / jnp.maximum(cntv[i, :], 1.0)
        for j in range(HALF // 16):
            accv[i, pl.ds(16 * j, 16)] = accv[i, pl.ds(16 * j, 16)] * inv
    pltpu.sync_copy(accv,
                    out_hbm.at[pl.ds(seg0, SEG_PER_TILE), pl.ds(col0, HALF)])


def kernel(x, batch):
    zeros = jnp.zeros((SEG_PER_TILE, HALF), jnp.float32)
    iota = jnp.arange(G, dtype=jnp.int32).reshape(CH, 128)
    return _pool_kernel(x, batch, zeros, iota)


# speedup vs baseline: 1.8692x; 1.0819x over previous
import functools

import jax
import jax.numpy as jnp
from jax import lax
from jax.experimental import pallas as pl
from jax.experimental.pallas import tpu as pltpu
from jax.experimental.pallas import tpu_sc as plsc

N = 100000
D = 128
G = 512

NC = 2
NS = 16
HALF = D // NC
R = 512
NFULL = N // R
TAIL = N - NFULL * R
NB = NFULL // NS
EXTRA = NFULL - NB * NS
SEG_PER_TILE = G // NS
CH = R // 128
NSTREAM = 4

_mesh = plsc.VectorSubcoreMesh(core_axis_name="c", subcore_axis_name="s")


@functools.partial(
    pl.kernel,
    out_type=jax.ShapeDtypeStruct((G, D), jnp.float32),
    mesh=_mesh,
    compiler_params=pltpu.CompilerParams(use_tc_tiling_on_sc=False,
                                         needs_layout_passes=False),
    scratch_types=[
        pltpu.VMEM((2, R, HALF), jnp.float32),
        pltpu.VMEM((2, CH, 128), jnp.int32),
        pltpu.VMEM((32,), jnp.int32),
        pltpu.VMEM((CH, 128), jnp.int32),
        pltpu.VMEM((G, HALF), jnp.float32),
        pltpu.VMEM((G, 16), jnp.float32),
        pltpu.VMEM((SEG_PER_TILE, HALF), jnp.float32),
        pltpu.VMEM((SEG_PER_TILE, 16), jnp.float32),
        pltpu.VMEM_SHARED((G, HALF), jnp.float32),
        pltpu.VMEM_SHARED((G, 16), jnp.float32),
        pltpu.SemaphoreType.DMA,
        pltpu.SemaphoreType.DMA,
        pltpu.SemaphoreType.DMA,
        pltpu.SemaphoreType.DMA,
        pltpu.SemaphoreType.DMA,
    ],
)
def _pool_kernel(x_hbm, b_hbm, z_hbm, iota_hbm, out_hbm,
                 xb, idxb, idx_t, iotav, accl, cntl, accv, cntv,
                 acc_sh, cnt_sh, sx0, sx1, si0, si1, ssc):
    c = lax.axis_index("c")
    s = lax.axis_index("s")
    col0 = c * HALF
    seg0 = s * SEG_PER_TILE
    sx = (sx0, sx1)
    si = (si0, si1)
    zvec = jnp.zeros((16,), jnp.float32)
    full16 = jnp.full((16,), 16.0, jnp.float32)
    ones16 = jnp.ones((16,), jnp.float32)

    def start_load(kblk, b):
        r0 = (s + NS * kblk) * R
        pltpu.async_copy(x_hbm.at[pl.ds(r0, R), pl.ds(col0, HALF)],
                         xb.at[b], sx[b])
        for m in range(CH):
            pltpu.async_copy(b_hbm.at[pl.ds(r0 + 128 * m, 128)],
                             idxb.at[b, m], si[b])

    def wait_load(b):
        pltpu.make_async_copy(x_hbm.at[pl.ds(0, R), pl.ds(0, HALF)],
                              xb.at[b], sx[b]).wait()
        for m in range(CH):
            pltpu.make_async_copy(b_hbm.at[pl.ds(0, 128)],
                                  idxb.at[b, m], si[b]).wait()

    def group_body(idvec, row0, b, xmode):
        sid0 = idvec[0]
        uniform = jnp.all(idvec == sid0)

        @pl.when(uniform)
        def _u():
            plsc.addupdate(cntl.at[sid0], full16)
            if xmode:
                for j in range(HALF // 16):
                    acc = xb[b, row0, pl.ds(16 * j, 16)]
                    for r in range(1, 16):
                        acc = acc + xb[b, row0 + r, pl.ds(16 * j, 16)]
                    plsc.addupdate(accl.at[sid0, pl.ds(16 * j, 16)], acc)

        @pl.when(jnp.logical_not(uniform))
        def _n():
            for r in range(16):
                sid = idvec[r]
                plsc.addupdate(cntl.at[sid], ones16)
                if xmode:
                    for j in range(HALF // 16):
                        xv = xb[b, row0 + r, pl.ds(16 * j, 16)]
                        plsc.addupdate(accl.at[sid, pl.ds(16 * j, 16)], xv)

    def process_buf(b):
        handles = [pltpu.async_copy(xb.at[b, pl.ds(128 * m, 128)],
                                    acc_sh.at[idxb.at[b, m]], ssc, add=True)
                   for m in range(NSTREAM)]
        for m in range(CH):
            xmode = m >= NSTREAM

            def grp(u, carry, b=b, m=m, xmode=xmode):
                idvec = idxb[b, m, pl.ds(16 * u, 16)]
                group_body(idvec, 128 * m + 16 * u, b, xmode)
                return carry

            lax.fori_loop(0, 8, grp, 0)
        for h in handles:
            h.wait()

    start_load(0, 0)
    start_load(1, 1)
    pltpu.sync_copy(z_hbm, acc_sh.at[pl.ds(seg0, SEG_PER_TILE)])
    pltpu.sync_copy(z_hbm.at[:, pl.ds(0, 16)],
                    cnt_sh.at[pl.ds(seg0, SEG_PER_TILE)])
    pltpu.sync_copy(iota_hbm, iotav)

    def zrow(i, carry):
        for j in range(HALF // 16):
            accl[i, pl.ds(16 * j, 16)] = zvec
        cntl[i, pl.ds(0, 16)] = zvec
        return carry

    lax.fori_loop(0, G, zrow, 0)
    plsc.subcore_barrier()

    def blk_body(g, carry):
        for b in range(2):
            wait_load(b)
            process_buf(b)
            start_load(2 * g + b + 2, b)
        return carry

    lax.fori_loop(0, NB // 2 - 1, blk_body, 0)
    for b in range(2):
        wait_load(b)
        process_buf(b)

    @pl.when(s < EXTRA)
    def _extra():
        r0 = (NB * NS + s) * R
        pltpu.sync_copy(x_hbm.at[pl.ds(r0, R), pl.ds(col0, HALF)], xb.at[0])
        for m in range(CH):
            pltpu.sync_copy(b_hbm.at[pl.ds(r0 + 128 * m, 128)], idxb.at[0, m])
        process_buf(0)

    @pl.when(s == EXTRA)
    def _tail():
        r0 = NFULL * R
        pltpu.sync_copy(x_hbm.at[pl.ds(r0, TAIL), pl.ds(col0, HALF)],
                        xb.at[0, pl.ds(0, TAIL)])
        pltpu.sync_copy(b_hbm.at[pl.ds(r0, 128)], idxb.at[0, 0])
        pltpu.sync_copy(b_hbm.at[pl.ds(r0 + 128, 32)], idx_t)

        def tgrp(u, carry):
            idvec = idxb[0, 0, pl.ds(16 * u, 16)]
            group_body(idvec, 16 * u, 0, True)
            return carry

        lax.fori_loop(0, 8, tgrp, 0)

        def tgrp2(u, carry):
            idvec = idx_t[pl.ds(16 * u, 16)]
            group_body(idvec, 128 + 16 * u, 0, True)
            return carry

        lax.fori_loop(0, 2, tgrp2, 0)

    handles = [pltpu.async_copy(accl.at[pl.ds(128 * q, 128)],
                                acc_sh.at[iotav.at[q]], ssc, add=True)
               for q in range(CH)]
    handles += [pltpu.async_copy(cntl.at[pl.ds(128 * q, 128)],
                                 cnt_sh.at[iotav.at[q]], ssc, add=True)
                for q in range(CH)]
    for h in handles:
        h.wait()
    plsc.subcore_barrier()

    pltpu.sync_copy(acc_sh.at[pl.ds(seg0, SEG_PER_TILE)], accv)
    pltpu.sync_copy(cnt_sh.at[pl.ds(seg0, SEG_PER_TILE)], cntv)
    for i in range(SEG_PER_TILE):
        inv = 1.0 / jnp.maximum(cntv[i, :], 1.0)
        for j in range(HALF // 16):
            accv[i, pl.ds(16 * j, 16)] = accv[i, pl.ds(16 * j, 16)] * inv
    pltpu.sync_copy(accv,
                    out_hbm.at[pl.ds(seg0, SEG_PER_TILE), pl.ds(col0, HALF)])


def kernel(x, batch):
    zeros = jnp.zeros((SEG_PER_TILE, HALF), jnp.float32)
    iota = jnp.arange(G, dtype=jnp.int32).reshape(CH, 128)
    return _pool_kernel(x, batch, zeros, iota)
